# baseline (speedup 1.0000x reference)
import functools

import jax
import jax.numpy as jnp
from jax.experimental import pallas as pl
from jax.experimental.pallas import tpu as pltpu

T = 2048
M = 2048
E = 8
H = 4096
C = 256
NKA = 4
NKB = 4
BM = M // NKA
BH = H // NKB
NS = NKA + NKB


def _moe_body(x_ref, ssrc_ref, sgate_ref, fc1_ref, b1_ref, fc2_ref, b2_ref,
              out_ref, xe_raw, xs_ref, hacc_ref, hb_ref, acc_ref, stage_ref,
              sem_in, sem_out, sem_z):
    e = pl.program_id(0)
    s = pl.program_id(1)

    def issue_gather(e1, b):
        def body(c, _):
            t1 = ssrc_ref[e1, c]
            t = jnp.maximum(t1 - 1, 0)
            pltpu.make_async_copy(
                x_ref.at[pl.ds(t, 1), :],
                xe_raw.at[b, pl.ds(c, 1), :],
                sem_in.at[b],
            ).start()
            return 0
        jax.lax.fori_loop(0, C, body, 0, unroll=False)

    @pl.when(s == 0)
    def _():
        b = jax.lax.rem(e, 2)

        @pl.when(e == 0)
        def _():
            issue_gather(0, 0)
            stage_ref[...] = jnp.zeros_like(stage_ref)
            def zbody(k, _):
                pltpu.make_async_copy(
                    stage_ref, out_ref.at[pl.ds(k * C, C), :], sem_z,
                ).start()
                return 0
            jax.lax.fori_loop(0, T // C, zbody, 0, unroll=False)

        @pl.when(e + 1 < E)
        def _():
            issue_gather(e + 1, jax.lax.rem(e + 1, 2))

        pltpu.make_async_copy(
            x_ref.at[pl.ds(0, C), :], xe_raw.at[b], sem_in.at[b]
        ).wait()
        xs_ref[...] = (xe_raw[b] * sgate_ref[0]).astype(jnp.bfloat16)

    @pl.when(s < NKA)
    def _():
        xs_blk = xs_ref[:, pl.ds(pl.multiple_of(s * BM, BM), BM)]
        contrib = jnp.dot(xs_blk, fc1_ref[0].astype(jnp.bfloat16),
                          preferred_element_type=jnp.float32)

        @pl.when(s == 0)
        def _():
            hacc_ref[...] = contrib

        @pl.when(s != 0)
        def _():
            hacc_ref[...] += contrib

        @pl.when(s == NKA - 1)
        def _():
            hb_ref[...] = jnp.maximum(
                hacc_ref[...] + b1_ref[0], 0.0).astype(jnp.bfloat16)

    @pl.when(s >= NKA)
    def _():
        k = s - NKA
        h_blk = hb_ref[:, pl.ds(pl.multiple_of(k * BH, BH), BH)]
        contrib = jnp.dot(h_blk, fc2_ref[0].astype(jnp.bfloat16),
                          preferred_element_type=jnp.float32)

        @pl.when(k == 0)
        def _():
            acc_ref[...] = contrib + b2_ref[0]

        @pl.when(k != 0)
        def _():
            acc_ref[...] += contrib

    @pl.when(s == NS - 1)
    def _():
        @pl.when(e == 0)
        def _():
            def zwait(k, _):
                pltpu.make_async_copy(
                    stage_ref, out_ref.at[pl.ds(k * C, C), :], sem_z,
                ).wait()
                return 0
            jax.lax.fori_loop(0, T // C, zwait, 0, unroll=False)

        stage_ref[...] = acc_ref[...] * sgate_ref[0]

        def sbody(c, cnt):
            t1 = ssrc_ref[e, c]

            def do_start():
                pltpu.make_async_copy(
                    stage_ref.at[pl.ds(c, 1), :],
                    out_ref.at[pl.ds(t1 - 1, 1), :],
                    sem_out,
                ).start()

            jax.lax.cond(t1 > 0, do_start, lambda: None)
            return cnt + jnp.where(t1 > 0, 1, 0)

        cnt = jax.lax.fori_loop(0, C, sbody, 0, unroll=False)

        def wbody(i, _):
            pltpu.make_async_copy(
                x_ref.at[pl.ds(0, 1), :],
                stage_ref.at[pl.ds(0, 1), :],
                sem_out,
            ).wait()
            return 0
        jax.lax.fori_loop(0, cnt, wbody, 0, unroll=False)


@jax.jit
def _moe(x, ssrc, sgate, fc1, b1, fc2, b2):
    return pl.pallas_call(
        _moe_body,
        grid=(E, NS),
        in_specs=[
            pl.BlockSpec(memory_space=pltpu.MemorySpace.HBM),
            pl.BlockSpec(memory_space=pltpu.SMEM),
            pl.BlockSpec((1, C, 1), lambda e, s: (e, 0, 0)),
            pl.BlockSpec((1, BM, H), lambda e, s: (e, jnp.minimum(s, NKA - 1), 0)),
            pl.BlockSpec((1, 1, H), lambda e, s: (e, 0, 0)),
            pl.BlockSpec((1, BH, M), lambda e, s: (e, jnp.maximum(s - NKA, 0), 0)),
            pl.BlockSpec((1, 1, M), lambda e, s: (e, 0, 0)),
        ],
        out_specs=pl.BlockSpec(memory_space=pltpu.MemorySpace.HBM),
        out_shape=jax.ShapeDtypeStruct((T, M), jnp.float32),
        scratch_shapes=[
            pltpu.VMEM((2, C, M), jnp.float32),
            pltpu.VMEM((C, M), jnp.bfloat16),
            pltpu.VMEM((C, H), jnp.float32),
            pltpu.VMEM((C, H), jnp.bfloat16),
            pltpu.VMEM((C, M), jnp.float32),
            pltpu.VMEM((C, M), jnp.float32),
            pltpu.SemaphoreType.DMA((2,)),
            pltpu.SemaphoreType.DMA,
            pltpu.SemaphoreType.DMA,
        ],
        compiler_params=pltpu.CompilerParams(
            dimension_semantics=("arbitrary", "arbitrary"),
        ),
    )(x, ssrc, sgate, fc1, b1, fc2, b2)


def kernel(x, wg, fc1, b1, fc2, b2):
    T_, M_ = x.shape
    E_ = wg.shape[0]

    logits = x @ wg.T
    indices1_s = jnp.argmax(logits, axis=1)
    mask1 = jax.nn.one_hot(indices1_s, E_, dtype=logits.dtype)
    gates = jax.nn.softmax(logits, axis=1)
    gates1_s = jnp.sum(gates * mask1, axis=1)
    locations = jnp.cumsum(mask1, axis=0) - mask1
    locations1_s = jnp.sum(locations * mask1, axis=1).astype(jnp.int32)

    valid = locations1_s < C
    pos = indices1_s.astype(jnp.int32) * C + locations1_s
    pos_scatter = jnp.where(valid, pos, E_ * C)
    tok1 = jnp.arange(1, T_ + 1, dtype=jnp.int32)
    ssrc = jnp.zeros((E_ * C + 1,), jnp.int32).at[pos_scatter].set(tok1)[:E_ * C]
    sgate = jnp.zeros((E_ * C + 1,), jnp.float32).at[pos_scatter].set(gates1_s)[:E_ * C]

    out = _moe(x, ssrc.reshape(E_, C), sgate.reshape(E_, C, 1),
               fc1, b1.reshape(E_, 1, H), fc2, b2.reshape(E_, 1, M))
    return out


# speedup vs baseline: 1.0872x; 1.0872x over previous
import functools

import jax
import jax.numpy as jnp
from jax.experimental import pallas as pl
from jax.experimental.pallas import tpu as pltpu

T = 2048
M = 2048
E = 8
H = 4096
C = 256
BH = 1024
NH = H // BH


def _moe_body(x_ref, ssrc_ref, sgate_ref, fc1_ref, b1_ref, fc2_ref, b2_ref,
              out_ref, xe_raw, xs_ref, acc_ref, stage_ref,
              sem_in, sem_out, sem_z):
    e = pl.program_id(0)
    nh = pl.program_id(1)

    def issue_gather(e1, b):
        def body(c, _):
            t1 = ssrc_ref[e1, c]
            t = jnp.maximum(t1 - 1, 0)
            pltpu.make_async_copy(
                x_ref.at[pl.ds(t, 1), :],
                xe_raw.at[b, pl.ds(c, 1), :],
                sem_in.at[b],
            ).start()
            return 0
        jax.lax.fori_loop(0, C, body, 0, unroll=False)

    @pl.when(nh == 0)
    def _():
        b = jax.lax.rem(e, 2)

        @pl.when(e == 0)
        def _():
            issue_gather(0, 0)
            stage_ref[...] = jnp.zeros_like(stage_ref)

            def zbody(k, _):
                pltpu.make_async_copy(
                    stage_ref, out_ref.at[pl.ds(k * C, C), :], sem_z,
                ).start()
                return 0
            jax.lax.fori_loop(0, T // C, zbody, 0, unroll=False)

        @pl.when(e + 1 < E)
        def _():
            issue_gather(e + 1, jax.lax.rem(e + 1, 2))

        pltpu.make_async_copy(
            x_ref.at[pl.ds(0, C), :], xe_raw.at[b], sem_in.at[b]
        ).wait()
        xs_ref[...] = (xe_raw[b] * sgate_ref[0]).astype(jnp.bfloat16)

    h = jnp.dot(xs_ref[...], fc1_ref[0].astype(jnp.bfloat16),
                preferred_element_type=jnp.float32)
    h = jnp.maximum(h + b1_ref[0], 0.0)
    contrib = jnp.dot(h.astype(jnp.bfloat16), fc2_ref[0].astype(jnp.bfloat16),
                      preferred_element_type=jnp.float32)

    @pl.when(nh == 0)
    def _():
        acc_ref[...] = contrib + b2_ref[0]

    @pl.when(nh != 0)
    def _():
        acc_ref[...] += contrib

    @pl.when(nh == NH - 1)
    def _():
        @pl.when(e == 0)
        def _():
            def zwait(k, _):
                pltpu.make_async_copy(
                    stage_ref, out_ref.at[pl.ds(k * C, C), :], sem_z,
                ).wait()
                return 0
            jax.lax.fori_loop(0, T // C, zwait, 0, unroll=False)

        stage_ref[...] = acc_ref[...] * sgate_ref[0]

        def sbody(c, cnt):
            t1 = ssrc_ref[e, c]

            def do_start():
                pltpu.make_async_copy(
                    stage_ref.at[pl.ds(c, 1), :],
                    out_ref.at[pl.ds(t1 - 1, 1), :],
                    sem_out,
                ).start()

            jax.lax.cond(t1 > 0, do_start, lambda: None)
            return cnt + jnp.where(t1 > 0, 1, 0)

        cnt = jax.lax.fori_loop(0, C, sbody, 0, unroll=False)

        def wbody(i, _):
            pltpu.make_async_copy(
                x_ref.at[pl.ds(0, 1), :],
                stage_ref.at[pl.ds(0, 1), :],
                sem_out,
            ).wait()
            return 0
        jax.lax.fori_loop(0, cnt, wbody, 0, unroll=False)


@jax.jit
def _moe(x, ssrc, sgate, fc1, b1, fc2, b2):
    return pl.pallas_call(
        _moe_body,
        grid=(E, NH),
        in_specs=[
            pl.BlockSpec(memory_space=pltpu.MemorySpace.HBM),
            pl.BlockSpec(memory_space=pltpu.SMEM),
            pl.BlockSpec((1, C, 1), lambda e, nh: (e, 0, 0)),
            pl.BlockSpec((1, M, BH), lambda e, nh: (e, 0, nh)),
            pl.BlockSpec((1, 1, BH), lambda e, nh: (e, 0, nh)),
            pl.BlockSpec((1, BH, M), lambda e, nh: (e, nh, 0)),
            pl.BlockSpec((1, 1, M), lambda e, nh: (e, 0, 0)),
        ],
        out_specs=pl.BlockSpec(memory_space=pltpu.MemorySpace.HBM),
        out_shape=jax.ShapeDtypeStruct((T, M), jnp.float32),
        scratch_shapes=[
            pltpu.VMEM((2, C, M), jnp.float32),
            pltpu.VMEM((C, M), jnp.bfloat16),
            pltpu.VMEM((C, M), jnp.float32),
            pltpu.VMEM((C, M), jnp.float32),
            pltpu.SemaphoreType.DMA((2,)),
            pltpu.SemaphoreType.DMA,
            pltpu.SemaphoreType.DMA,
        ],
        compiler_params=pltpu.CompilerParams(
            dimension_semantics=("arbitrary", "arbitrary"),
        ),
    )(x, ssrc, sgate, fc1, b1, fc2, b2)


def kernel(x, wg, fc1, b1, fc2, b2):
    T_, M_ = x.shape
    E_ = wg.shape[0]

    logits = x @ wg.T
    indices1_s = jnp.argmax(logits, axis=1)
    mask1 = jax.nn.one_hot(indices1_s, E_, dtype=logits.dtype)
    gates = jax.nn.softmax(logits, axis=1)
    gates1_s = jnp.sum(gates * mask1, axis=1)
    locations = jnp.cumsum(mask1, axis=0) - mask1
    locations1_s = jnp.sum(locations * mask1, axis=1).astype(jnp.int32)

    valid = locations1_s < C
    pos = indices1_s.astype(jnp.int32) * C + locations1_s
    pos_scatter = jnp.where(valid, pos, E_ * C)
    tok1 = jnp.arange(1, T_ + 1, dtype=jnp.int32)
    ssrc = jnp.zeros((E_ * C + 1,), jnp.int32).at[pos_scatter].set(tok1)[:E_ * C]
    sgate = jnp.zeros((E_ * C + 1,), jnp.float32).at[pos_scatter].set(gates1_s)[:E_ * C]

    out = _moe(x, ssrc.reshape(E_, C), sgate.reshape(E_, C, 1),
               fc1, b1.reshape(E_, 1, H), fc2, b2.reshape(E_, 1, M))
    return out
